# baseline (speedup 1.0000x reference)
import jax
import jax.numpy as jnp
from jax import lax
from jax.experimental import pallas as pl
from jax.experimental.pallas import tpu as pltpu
from jax.experimental.pallas import tpu_sc as plsc

BSZ = 4096
MSL = 200
VOCAB = 1000000
EMBDIM = 32

NC = 2
NS = 16
NW = NC * NS
B_PER_W = BSZ // NW
CB = 4
NCHUNK = B_PER_W // CB
HALF = MSL // 2

LANE = 128
NBLK = VOCAB // LANE
TAIL = VOCAB - NBLK * LANE
EW = EMBDIM + 1


def _mesh():
    return plsc.VectorSubcoreMesh(
        core_axis_name="c", subcore_axis_name="s", num_cores=NC, num_subcores=NS
    )


def _wid():
    return lax.axis_index("s") * NC + lax.axis_index("c")


def _reformat_body(tableT, inputsT, tail_lin, tbl_out, idx_out,
                   tin0, tin1, tout0, tout1, iin_v, iout_v, tail_v,
                   si0, si1, so0, so1):
    wid = _wid()
    lanes = lax.broadcasted_iota(jnp.int32, (16,), 0)
    t_idx = [lanes * EW + 16 * EW * j for j in range(8)]
    i_idx = [lanes * MSL + 16 * MSL * j for j in range(8)]

    pltpu.sync_copy(inputsT.at[:, pl.ds(wid * LANE, LANE)], iin_v)

    @plsc.parallel_loop(0, MSL, unroll=2)
    def _(d):
        for j in range(8):
            x = iin_v[d, pl.ds(16 * j, 16)]
            plsc.store_scatter(iout_v, [i_idx[j] + d], x)

    pltpu.sync_copy(iout_v, idx_out.at[pl.ds(wid * B_PER_W * MSL, B_PER_W * MSL)])

    tin = (tin0, tin1)
    tout = (tout0, tout1)
    si = (si0, si1)
    so = (so0, so1)
    UNIF = NBLK // NW

    def in_desc(c, db):
        tc = wid + c * NW
        return pltpu.make_async_copy(
            tableT.at[:, pl.ds(tc * LANE, LANE)], tin[db], si[db]
        )

    def out_desc(c, db):
        tc = wid + c * NW
        return pltpu.make_async_copy(
            tout[db], tbl_out.at[pl.ds(tc * LANE * EW, LANE * EW)], so[db]
        )

    def scatter(db):
        @plsc.parallel_loop(0, EMBDIM, unroll=4)
        def _(d):
            for j in range(8):
                x = tin[db][d, pl.ds(16 * j, 16)]
                plsc.store_scatter(tout[db], [t_idx[j] + d], x)

    in_desc(0, 0).start()
    in_desc(1, 1).start()
    for db in (0, 1):
        in_desc(db, db).wait()
        scatter(db)
        out_desc(db, db).start()
        in_desc(db + 2, db).start()

    def step(i, carry):
        for db in (0, 1):
            c = 2 * i + db
            in_desc(c, db).wait()
            out_desc(c - 2, db).wait()
            scatter(db)
            out_desc(c, db).start()
            in_desc(c + 2, db).start()
        return carry

    lax.fori_loop(1, UNIF // 2 - 1, step, 0)
    for db in (0, 1):
        c = UNIF - 2 + db
        in_desc(c, db).wait()
        out_desc(c - 2, db).wait()
        scatter(db)
        out_desc(c, db).start()
    out_desc(UNIF - 2, 0).wait()
    out_desc(UNIF - 1, 1).wait()

    @pl.when(wid < NBLK - UNIF * NW)
    def _():
        tc = UNIF * NW + wid
        pltpu.sync_copy(tableT.at[:, pl.ds(tc * LANE, LANE)], tin0)
        scatter(0)
        pltpu.sync_copy(
            tout0, tbl_out.at[pl.ds(tc * LANE * EW, LANE * EW)]
        )

    @pl.when(wid == NW - 1)
    def _():
        pltpu.sync_copy(tail_lin, tail_v)

        def tl(v, carry):
            x0 = tail_v[pl.ds(EMBDIM * v, 16)]
            x1 = tail_v[pl.ds(EMBDIM * v + 16, 16)]
            plsc.store_scatter(tout0, [lanes + EW * v], x0)
            plsc.store_scatter(tout0, [lanes + (EW * v + 16)], x1)
            return carry

        lax.fori_loop(0, TAIL, tl, 0)
        pltpu.sync_copy(
            tout0.at[pl.ds(0, TAIL * EW)],
            tbl_out.at[pl.ds(NBLK * LANE * EW, TAIL * EW)],
        )


def _reformat(tableT, inputsT, tail_lin):
    return pl.kernel(
        _reformat_body,
        out_type=(
            jax.ShapeDtypeStruct((VOCAB * EW,), jnp.float32),
            jax.ShapeDtypeStruct((BSZ * MSL,), jnp.int32),
        ),
        mesh=_mesh(),
        scratch_types=[
            pltpu.VMEM((EMBDIM, LANE), jnp.float32),
            pltpu.VMEM((EMBDIM, LANE), jnp.float32),
            pltpu.VMEM((LANE * EW,), jnp.float32),
            pltpu.VMEM((LANE * EW,), jnp.float32),
            pltpu.VMEM((MSL, LANE), jnp.int32),
            pltpu.VMEM((LANE * MSL,), jnp.int32),
            pltpu.VMEM((TAIL * EMBDIM,), jnp.float32),
            pltpu.SemaphoreType.DMA,
            pltpu.SemaphoreType.DMA,
            pltpu.SemaphoreType.DMA,
            pltpu.SemaphoreType.DMA,
        ],
        compiler_params=pltpu.CompilerParams(
            use_tc_tiling_on_sc=True, needs_layout_passes=False
        ),
    )(tableT, inputsT, tail_lin)


def _gather_body(idx_hbm, table_hbm, out_hbm, idx_v, rows_v, out_v, sem0, sem1):
    wid = _wid()
    b_base = wid * B_PER_W
    pltpu.sync_copy(idx_hbm.at[pl.ds(b_base, B_PER_W)], idx_v)
    sems = (sem0, sem1)

    def fire(c, db):
        for b in range(CB):
            pltpu.async_copy(
                table_hbm.at[idx_v.at[c * CB + b]], rows_v.at[db, b], sems[db]
            )

    def drain(c, db):
        for b in range(CB):
            pltpu.make_async_copy(
                table_hbm.at[idx_v.at[c * CB + b]], rows_v.at[db, b], sems[db]
            ).wait()

    def reduce(c, db):
        zero = jnp.zeros((16,), jnp.float32)
        for b in range(CB):

            @plsc.parallel_loop(0, HALF, unroll=4, carry=(zero, zero, zero, zero))
            def accs(j, carry):
                c00, c01, c10, c11 = carry
                c00 = c00 + rows_v[db, b, j, 0:16]
                c01 = c01 + rows_v[db, b, j, 16:32]
                c10 = c10 + rows_v[db, b, j + HALF, 0:16]
                c11 = c11 + rows_v[db, b, j + HALF, 16:32]
                return c00, c01, c10, c11

            c00, c01, c10, c11 = accs
            out_v[c * CB + b, 0:16] = c00 + c10
            out_v[c * CB + b, 16:32] = c01 + c11

    fire(0, 0)
    fire(1, 1)

    def step(i, carry):
        c0 = 2 * i
        drain(c0, 0)
        reduce(c0, 0)
        fire(c0 + 2, 0)
        drain(c0 + 1, 1)
        reduce(c0 + 1, 1)
        fire(c0 + 3, 1)
        return carry

    lax.fori_loop(0, NCHUNK // 2 - 1, step, 0)
    drain(NCHUNK - 2, 0)
    reduce(NCHUNK - 2, 0)
    drain(NCHUNK - 1, 1)
    reduce(NCHUNK - 1, 1)
    pltpu.sync_copy(out_v, out_hbm.at[pl.ds(b_base, B_PER_W)])


def _gather(idx, table):
    return pl.kernel(
        _gather_body,
        out_type=jax.ShapeDtypeStruct((BSZ, EMBDIM), jnp.float32),
        mesh=_mesh(),
        scratch_types=[
            pltpu.VMEM((B_PER_W, MSL), jnp.int32),
            pltpu.VMEM((2, CB, MSL, EW), jnp.float32),
            pltpu.VMEM((B_PER_W, EMBDIM), jnp.float32),
            pltpu.SemaphoreType.DMA,
            pltpu.SemaphoreType.DMA,
        ],
        compiler_params=pltpu.CompilerParams(use_tc_tiling_on_sc=False),
    )(idx, table)


@jax.jit
def kernel(inputs, table):
    tail_lin = table[NBLK * LANE:].reshape(-1)
    tbl_lin, idx_lin = _reformat(table.T, inputs.T, tail_lin)
    return _gather(idx_lin.reshape(BSZ, MSL), tbl_lin.reshape(VOCAB, EW))


# speedup vs baseline: 6.6904x; 6.6904x over previous
import jax
import jax.numpy as jnp
from jax import lax
from jax.experimental import pallas as pl
from jax.experimental.pallas import tpu as pltpu
from jax.experimental.pallas import tpu_sc as plsc

BSZ = 4096
MSL = 200
VOCAB = 1000000
EMBDIM = 32

NC = 2
NS = 16
NW = NC * NS
B_PER_W = BSZ // NW
CB = 4
NCHUNK = B_PER_W // CB
HALF = MSL // 2

LANE = 128
NBLK = VOCAB // LANE
TAIL = VOCAB - NBLK * LANE


def _mesh():
    return plsc.VectorSubcoreMesh(
        core_axis_name="c", subcore_axis_name="s", num_cores=NC, num_subcores=NS
    )


def _wid():
    return lax.axis_index("s") * NC + lax.axis_index("c")


def _reformat_body(tableT, inputsT, tail_lin, tbl_out, idx_out,
                   tin0, tin1, tout0, tout1, iin_v, iout_v,
                   si0, si1, so0, so1):
    wid = _wid()
    lanes = lax.broadcasted_iota(jnp.int32, (16,), 0)
    i_idx = [lanes * MSL + 16 * MSL * j for j in range(8)]

    pltpu.sync_copy(inputsT.at[:, pl.ds(wid * LANE, LANE)], iin_v)

    @plsc.parallel_loop(0, MSL, unroll=2)
    def _(d):
        for j in range(8):
            x = iin_v[d, pl.ds(16 * j, 16)]
            plsc.store_scatter(iout_v, [i_idx[j] + d], x)

    pltpu.sync_copy(iout_v, idx_out.at[pl.ds(wid * B_PER_W * MSL, B_PER_W * MSL)])

    tin = (tin0, tin1)
    tout = (tout0, tout1)
    si = (si0, si1)
    so = (so0, so1)
    UNIF = NBLK // NW

    def in_desc(c, db):
        tc = wid + c * NW
        return pltpu.make_async_copy(
            tableT.at[:, pl.ds(tc * LANE, LANE)], tin[db], si[db]
        )

    def out_desc(c, db):
        tc = wid + c * NW
        return pltpu.make_async_copy(
            tout[db], tbl_out.at[pl.ds(tc * LANE * EMBDIM, LANE * EMBDIM)], so[db]
        )

    def scatter(db):
        @plsc.parallel_loop(0, 16, unroll=2)
        def _(c):
            rot = (lanes + c) & 15
            svec = (lanes << 5) + rot
            for h in (0, 1):
                dvec = rot + 16 * h
                for j in range(8):
                    x = plsc.load_gather(tin[db], [dvec, lanes + 16 * j])
                    plsc.store_scatter(
                        tout[db], [svec + (512 * j + 16 * h)], x
                    )

    in_desc(0, 0).start()
    in_desc(1, 1).start()
    for db in (0, 1):
        in_desc(db, db).wait()
        scatter(db)
        out_desc(db, db).start()
        in_desc(db + 2, db).start()

    def step(i, carry):
        for db in (0, 1):
            c = 2 * i + db
            in_desc(c, db).wait()
            out_desc(c - 2, db).wait()
            scatter(db)
            out_desc(c, db).start()
            in_desc(c + 2, db).start()
        return carry

    lax.fori_loop(1, UNIF // 2 - 1, step, 0)
    for db in (0, 1):
        c = UNIF - 2 + db
        in_desc(c, db).wait()
        out_desc(c - 2, db).wait()
        scatter(db)
        out_desc(c, db).start()
    out_desc(UNIF - 2, 0).wait()
    out_desc(UNIF - 1, 1).wait()

    @pl.when(wid < NBLK - UNIF * NW)
    def _():
        tc = UNIF * NW + wid
        pltpu.sync_copy(tableT.at[:, pl.ds(tc * LANE, LANE)], tin0)
        scatter(0)
        pltpu.sync_copy(
            tout0, tbl_out.at[pl.ds(tc * LANE * EMBDIM, LANE * EMBDIM)]
        )

    @pl.when(wid == NW - 1)
    def _():
        pltpu.sync_copy(tail_lin, tout0.at[pl.ds(0, TAIL * EMBDIM)])
        pltpu.sync_copy(
            tout0.at[pl.ds(0, TAIL * EMBDIM)],
            tbl_out.at[pl.ds(NBLK * LANE * EMBDIM, TAIL * EMBDIM)],
        )


def _reformat(tableT, inputsT, tail_lin):
    return pl.kernel(
        _reformat_body,
        out_type=(
            jax.ShapeDtypeStruct((VOCAB * EMBDIM,), jnp.float32),
            jax.ShapeDtypeStruct((BSZ * MSL,), jnp.int32),
        ),
        mesh=_mesh(),
        scratch_types=[
            pltpu.VMEM((EMBDIM, LANE), jnp.float32),
            pltpu.VMEM((EMBDIM, LANE), jnp.float32),
            pltpu.VMEM((LANE * EMBDIM,), jnp.float32),
            pltpu.VMEM((LANE * EMBDIM,), jnp.float32),
            pltpu.VMEM((MSL, LANE), jnp.int32),
            pltpu.VMEM((LANE * MSL,), jnp.int32),
            pltpu.SemaphoreType.DMA,
            pltpu.SemaphoreType.DMA,
            pltpu.SemaphoreType.DMA,
            pltpu.SemaphoreType.DMA,
        ],
        compiler_params=pltpu.CompilerParams(
            use_tc_tiling_on_sc=True, needs_layout_passes=False
        ),
    )(tableT, inputsT, tail_lin)


def _gather_body(idx_hbm, table_hbm, out_hbm, idx_v, rows_v, out_v, sem0, sem1):
    wid = _wid()
    b_base = wid * B_PER_W
    pltpu.sync_copy(idx_hbm.at[pl.ds(b_base, B_PER_W)], idx_v)
    sems = (sem0, sem1)

    def fire(c, db):
        for b in range(CB):
            pltpu.async_copy(
                table_hbm.at[idx_v.at[c * CB + b]], rows_v.at[db, b], sems[db]
            )

    def drain(c, db):
        for b in range(CB):
            pltpu.make_async_copy(
                table_hbm.at[idx_v.at[c * CB + b]], rows_v.at[db, b], sems[db]
            ).wait()

    def reduce(c, db):
        zero = jnp.zeros((16,), jnp.float32)
        for b in range(CB):

            @plsc.parallel_loop(0, HALF, unroll=4, carry=(zero, zero, zero, zero))
            def accs(j, carry):
                c00, c01, c10, c11 = carry
                c00 = c00 + rows_v[db, b, j, 0:16]
                c01 = c01 + rows_v[db, b, j, 16:32]
                c10 = c10 + rows_v[db, b, j + HALF, 0:16]
                c11 = c11 + rows_v[db, b, j + HALF, 16:32]
                return c00, c01, c10, c11

            c00, c01, c10, c11 = accs
            out_v[c * CB + b, 0:16] = c00 + c10
            out_v[c * CB + b, 16:32] = c01 + c11

    fire(0, 0)
    fire(1, 1)

    def step(i, carry):
        c0 = 2 * i
        drain(c0, 0)
        reduce(c0, 0)
        fire(c0 + 2, 0)
        drain(c0 + 1, 1)
        reduce(c0 + 1, 1)
        fire(c0 + 3, 1)
        return carry

    lax.fori_loop(0, NCHUNK // 2 - 1, step, 0)
    drain(NCHUNK - 2, 0)
    reduce(NCHUNK - 2, 0)
    drain(NCHUNK - 1, 1)
    reduce(NCHUNK - 1, 1)
    pltpu.sync_copy(out_v, out_hbm.at[pl.ds(b_base, B_PER_W)])


def _gather(idx, table):
    return pl.kernel(
        _gather_body,
        out_type=jax.ShapeDtypeStruct((BSZ, EMBDIM), jnp.float32),
        mesh=_mesh(),
        scratch_types=[
            pltpu.VMEM((B_PER_W, MSL), jnp.int32),
            pltpu.VMEM((2, CB, MSL, EMBDIM), jnp.float32),
            pltpu.VMEM((B_PER_W, EMBDIM), jnp.float32),
            pltpu.SemaphoreType.DMA,
            pltpu.SemaphoreType.DMA,
        ],
        compiler_params=pltpu.CompilerParams(use_tc_tiling_on_sc=False),
    )(idx, table)


@jax.jit
def kernel(inputs, table):
    tail_lin = table[NBLK * LANE:].reshape(-1)
    tbl_lin, idx_lin = _reformat(table.T, inputs.T, tail_lin)
    return _gather(idx_lin.reshape(BSZ, MSL), tbl_lin.reshape(VOCAB, EMBDIM))


# speedup vs baseline: 8.6465x; 1.2924x over previous
import jax
import jax.numpy as jnp
from jax import lax
from jax.experimental import pallas as pl
from jax.experimental.pallas import tpu as pltpu
from jax.experimental.pallas import tpu_sc as plsc

BSZ = 4096
MSL = 200
VOCAB = 1000000
EMBDIM = 32

NC = 2
NS = 16
NW = NC * NS
B_PER_W = BSZ // NW
CB = 4
NCHUNK = B_PER_W // CB
HALF = MSL // 2

LANE = 128
BW = 128
NBLK = VOCAB // BW
TAIL = VOCAB - NBLK * BW


def _mesh():
    return plsc.VectorSubcoreMesh(
        core_axis_name="c", subcore_axis_name="s", num_cores=NC, num_subcores=NS
    )


def _wid():
    return lax.axis_index("s") * NC + lax.axis_index("c")


def _reformat_body(tableT, inputsT, tail_lin, tbl_out, idx_out,
                   tin0, tin1, tin2, tin3, tout0, tout1, tout2, tout3,
                   iin_v, iout_v, si0, si1, si2, si3, so0, so1, so2, so3):
    wid = _wid()
    lanes = lax.broadcasted_iota(jnp.int32, (16,), 0)
    i_idx = [lanes * MSL + 16 * MSL * j for j in range(8)]

    pltpu.sync_copy(inputsT.at[:, pl.ds(wid * LANE, LANE)], iin_v)

    @plsc.parallel_loop(0, MSL, unroll=2)
    def _(d):
        for j in range(8):
            x = iin_v[d, pl.ds(16 * j, 16)]
            plsc.store_scatter(iout_v, [i_idx[j] + d], x)

    pltpu.sync_copy(iout_v, idx_out.at[pl.ds(wid * B_PER_W * MSL, B_PER_W * MSL)])

    tin = (tin0, tin1, tin2, tin3)
    tout = (tout0, tout1, tout2, tout3)
    si = (si0, si1, si2, si3)
    so = (so0, so1, so2, so3)
    D = 4
    UNIF = NBLK // NW

    def in_desc(c, db):
        tc = wid + c * NW
        return pltpu.make_async_copy(
            tableT.at[:, pl.ds(tc * BW, BW)], tin[db], si[db]
        )

    def out_desc(c, db):
        tc = wid + c * NW
        return pltpu.make_async_copy(
            tout[db], tbl_out.at[pl.ds(tc * BW * EMBDIM, BW * EMBDIM)], so[db]
        )

    def scatter(db):
        @plsc.parallel_loop(0, 16, unroll=2)
        def _(c):
            rot = (lanes + c) & 15
            svec = (lanes << 5) + rot
            for h in (0, 1):
                dvec = rot + 16 * h
                for j in range(BW // 16):
                    x = plsc.load_gather(tin[db], [dvec, lanes + 16 * j])
                    plsc.store_scatter(
                        tout[db], [svec + (512 * j + 16 * h)], x
                    )

    for db in range(D):
        in_desc(db, db).start()
    for db in range(D):
        in_desc(db, db).wait()
        scatter(db)
        out_desc(db, db).start()
        in_desc(db + D, db).start()

    def step(i, carry):
        for db in range(D):
            c = D * i + db
            in_desc(c, db).wait()
            out_desc(c - D, db).wait()
            scatter(db)
            out_desc(c, db).start()
            in_desc(c + D, db).start()
        return carry

    lax.fori_loop(1, UNIF // D - 1, step, 0)
    for db in range(D):
        c = UNIF - D + db
        in_desc(c, db).wait()
        out_desc(c - D, db).wait()
        scatter(db)
        out_desc(c, db).start()
    for db in range(D):
        out_desc(UNIF - D + db, db).wait()

    @pl.when(wid < NBLK - UNIF * NW)
    def _():
        tc = UNIF * NW + wid
        pltpu.sync_copy(tableT.at[:, pl.ds(tc * BW, BW)], tin0)
        scatter(0)
        pltpu.sync_copy(
            tout0, tbl_out.at[pl.ds(tc * BW * EMBDIM, BW * EMBDIM)]
        )

    @pl.when(wid == NW - 1)
    def _():
        pltpu.sync_copy(tail_lin, tout0.at[pl.ds(0, TAIL * EMBDIM)])
        pltpu.sync_copy(
            tout0.at[pl.ds(0, TAIL * EMBDIM)],
            tbl_out.at[pl.ds(NBLK * BW * EMBDIM, TAIL * EMBDIM)],
        )


def _reformat(tableT, inputsT, tail_lin):
    return pl.kernel(
        _reformat_body,
        out_type=(
            jax.ShapeDtypeStruct((VOCAB * EMBDIM,), jnp.float32),
            jax.ShapeDtypeStruct((BSZ * MSL,), jnp.int32),
        ),
        mesh=_mesh(),
        scratch_types=[
            pltpu.VMEM((EMBDIM, BW), jnp.float32),
            pltpu.VMEM((EMBDIM, BW), jnp.float32),
            pltpu.VMEM((EMBDIM, BW), jnp.float32),
            pltpu.VMEM((EMBDIM, BW), jnp.float32),
            pltpu.VMEM((BW * EMBDIM,), jnp.float32),
            pltpu.VMEM((BW * EMBDIM,), jnp.float32),
            pltpu.VMEM((BW * EMBDIM,), jnp.float32),
            pltpu.VMEM((BW * EMBDIM,), jnp.float32),
            pltpu.VMEM((MSL, LANE), jnp.int32),
            pltpu.VMEM((LANE * MSL,), jnp.int32),
        ] + [pltpu.SemaphoreType.DMA] * 8,
        compiler_params=pltpu.CompilerParams(
            use_tc_tiling_on_sc=True, needs_layout_passes=False
        ),
    )(tableT, inputsT, tail_lin)


def _gather_body(idx_hbm, table_hbm, out_hbm, idx_v, rows_v, out_v, sem0, sem1):
    wid = _wid()
    b_base = wid * B_PER_W
    pltpu.sync_copy(idx_hbm.at[pl.ds(b_base, B_PER_W)], idx_v)
    sems = (sem0, sem1)

    def fire(c, db):
        for b in range(CB):
            pltpu.async_copy(
                table_hbm.at[idx_v.at[c * CB + b]], rows_v.at[db, b], sems[db]
            )

    def drain(c, db):
        for b in range(CB):
            pltpu.make_async_copy(
                table_hbm.at[idx_v.at[c * CB + b]], rows_v.at[db, b], sems[db]
            ).wait()

    def reduce(c, db):
        zero = jnp.zeros((16,), jnp.float32)
        for b in range(CB):

            @plsc.parallel_loop(0, HALF, unroll=4, carry=(zero, zero, zero, zero))
            def accs(j, carry):
                c00, c01, c10, c11 = carry
                c00 = c00 + rows_v[db, b, j, 0:16]
                c01 = c01 + rows_v[db, b, j, 16:32]
                c10 = c10 + rows_v[db, b, j + HALF, 0:16]
                c11 = c11 + rows_v[db, b, j + HALF, 16:32]
                return c00, c01, c10, c11

            c00, c01, c10, c11 = accs
            out_v[c * CB + b, 0:16] = c00 + c10
            out_v[c * CB + b, 16:32] = c01 + c11

    fire(0, 0)
    fire(1, 1)

    def step(i, carry):
        c0 = 2 * i
        drain(c0, 0)
        reduce(c0, 0)
        fire(c0 + 2, 0)
        drain(c0 + 1, 1)
        reduce(c0 + 1, 1)
        fire(c0 + 3, 1)
        return carry

    lax.fori_loop(0, NCHUNK // 2 - 1, step, 0)
    drain(NCHUNK - 2, 0)
    reduce(NCHUNK - 2, 0)
    drain(NCHUNK - 1, 1)
    reduce(NCHUNK - 1, 1)
    pltpu.sync_copy(out_v, out_hbm.at[pl.ds(b_base, B_PER_W)])


def _gather(idx, table):
    return pl.kernel(
        _gather_body,
        out_type=jax.ShapeDtypeStruct((BSZ, EMBDIM), jnp.float32),
        mesh=_mesh(),
        scratch_types=[
            pltpu.VMEM((B_PER_W, MSL), jnp.int32),
            pltpu.VMEM((2, CB, MSL, EMBDIM), jnp.float32),
            pltpu.VMEM((B_PER_W, EMBDIM), jnp.float32),
            pltpu.SemaphoreType.DMA,
            pltpu.SemaphoreType.DMA,
        ],
        compiler_params=pltpu.CompilerParams(use_tc_tiling_on_sc=False),
    )(idx, table)


@jax.jit
def kernel(inputs, table):
    tail_lin = table[NBLK * LANE:].reshape(-1)
    tbl_lin, idx_lin = _reformat(table.T, inputs.T, tail_lin)
    return _gather(idx_lin.reshape(BSZ, MSL), tbl_lin.reshape(VOCAB, EMBDIM))
